# Initial kernel scaffold; baseline (speedup 1.0000x reference)
import jax
import jax.numpy as jnp
from jax.experimental import pallas as pl


def kernel(x):
    raise NotImplementedError("write your pallas kernel here")


# speedup vs baseline: 8.4370x; 8.4370x over previous
import jax
import jax.numpy as jnp
from jax.experimental import pallas as pl

_SEQ = 200
_ROWS_PER_BLOCK = 16


def _entropy_block(x_ref, o_ref):
    xb = x_ref[...]
    eq = (xb[:, :, None] == xb[:, None, :]).astype(jnp.float32)
    counts = jnp.sum(eq, axis=-1)
    s = jnp.sum(jnp.log(counts), axis=-1)
    ent = jnp.log(jnp.float32(_SEQ)) - s * (1.0 / _SEQ)
    o_ref[...] = ent[:, None]


def kernel(x):
    b, s = x.shape
    r = _ROWS_PER_BLOCK
    return pl.pallas_call(
        _entropy_block,
        grid=(b // r,),
        in_specs=[pl.BlockSpec((r, s), lambda i: (i, 0))],
        out_specs=pl.BlockSpec((r, 1), lambda i: (i, 0)),
        out_shape=jax.ShapeDtypeStruct((b, 1), jnp.float32),
    )(x)


# speedup vs baseline: 46.0030x; 5.4525x over previous
import functools
import math

import jax
import jax.numpy as jnp
from jax import lax
from jax.experimental import pallas as pl
from jax.experimental.pallas import tpu as pltpu
from jax.experimental.pallas import tpu_sc as plsc

_VOCAB = 1000
_SEQ = 200
_BATCH = 16384
_NW = 32
_RPT = _BATCH // _NW
_GROUPS = _RPT // 16
_HIST_BINS = 1024
_LOG_SEQ = math.log(float(_SEQ))


def _entropy_sc(x_hbm, tab_hbm, out_hbm, tokens_v, tab_v, hist_v, out_v):
    wid = lax.axis_index("s") * 2 + lax.axis_index("c")
    base = wid * _RPT
    pltpu.sync_copy(x_hbm.at[pl.ds(base * _SEQ, _RPT * _SEQ)], tokens_v)
    pltpu.sync_copy(tab_hbm, tab_v)

    lane = lax.iota(jnp.int32, 16)
    zeros_i = jnp.zeros((16,), jnp.int32)
    ones_i = jnp.ones((16,), jnp.int32)
    lane_hist = lane * _HIST_BINS
    lane_row = lane * _SEQ

    def zero_hist(k, carry):
        hist_v[pl.ds(k * 16, 16)] = zeros_i
        return carry

    lax.fori_loop(0, (16 * _HIST_BINS) // 16, zero_hist, 0)

    def per_group(g, carry):
        tok_base = g * (16 * _SEQ) + lane_row

        def count_pass(t, c2):
            tok = plsc.load_gather(tokens_v, [tok_base + t])
            plsc.addupdate_scatter(hist_v, [lane_hist + tok], ones_i)
            return c2

        lax.fori_loop(0, _SEQ, count_pass, 0)

        def reduce_pass(t, acc):
            tok = plsc.load_gather(tokens_v, [tok_base + t])
            bins = lane_hist + tok
            cnt = plsc.load_gather(hist_v, [bins])
            plsc.store_scatter(hist_v, [bins], zeros_i)
            acc = acc + plsc.load_gather(tab_v, [cnt])
            return acc

        acc = lax.fori_loop(0, _SEQ, reduce_pass, jnp.zeros((16,), jnp.float32))
        out_v[pl.ds(g * 16, 16)] = _LOG_SEQ - acc * (1.0 / _SEQ)
        return carry

    lax.fori_loop(0, _GROUPS, per_group, 0)
    pltpu.sync_copy(out_v, out_hbm.at[pl.ds(base, _RPT)])


def kernel(x):
    c = jnp.arange(256, dtype=jnp.float32)
    tab = jnp.where(c > 0, c * jnp.log(jnp.maximum(c, 1.0)), 0.0)
    mesh = plsc.VectorSubcoreMesh(core_axis_name="c", subcore_axis_name="s")
    run = functools.partial(
        pl.kernel,
        mesh=mesh,
        out_type=jax.ShapeDtypeStruct((_BATCH,), jnp.float32),
        scratch_types=[
            pltpu.VMEM((_RPT * _SEQ,), jnp.int32),
            pltpu.VMEM((256,), jnp.float32),
            pltpu.VMEM((16 * _HIST_BINS,), jnp.int32),
            pltpu.VMEM((_RPT,), jnp.float32),
        ],
        compiler_params=pltpu.CompilerParams(needs_layout_passes=False),
    )(_entropy_sc)
    return run(x.reshape(-1), tab)[:, None]
